# baseline (speedup 1.0000x reference)
import jax
import jax.numpy as jnp
from jax.experimental import pallas as pl
from jax.experimental.pallas import tpu as pltpu

_DIM = 4096
_E = 64
_K = 8
_BLOCK = 1024
_KCH = 4
_CDIM = _DIM // _KCH


def _gate_block(x_ref, w_ref, wout_ref, iout_ref, acc_ref):
    j = pl.program_id(1)
    part = jax.lax.dot_general(
        x_ref[...], w_ref[...], (((1,), (1,)), ((), ())),
        preferred_element_type=jnp.float32)

    @pl.when(j == 0)
    def _():
        acc_ref[...] = part

    @pl.when(j > 0)
    def _():
        acc_ref[...] += part

    @pl.when(j == _KCH - 1)
    def _():
        scores = acc_ref[...]
        m = jnp.max(scores, axis=1, keepdims=True)
        e = jnp.exp(scores - m)
        denom = jnp.sum(e, axis=1, keepdims=True)

        lane = jax.lax.broadcasted_iota(jnp.int32, e.shape, 1)
        bits = jax.lax.bitcast_convert_type(e, jnp.int32)
        key = jax.lax.bitcast_convert_type(
            (bits & ~(_E - 1)) | ((_E - 1) - lane), jnp.float32)

        out_lane = jax.lax.broadcasted_iota(jnp.int32, (e.shape[0], _K), 1)
        wout = jnp.zeros((e.shape[0], _K), jnp.float32)
        iout = jnp.zeros((e.shape[0], _K), jnp.int32)
        work = key
        for k in range(_K):
            cur = jnp.max(work, axis=1, keepdims=True)
            cur_bits = jax.lax.bitcast_convert_type(cur, jnp.int32)
            idx = (_E - 1) - (cur_bits & (_E - 1))
            val = jax.lax.bitcast_convert_type(
                cur_bits & ~(_E - 1), jnp.float32)
            wout = jnp.where(out_lane == k, val, wout)
            iout = jnp.where(out_lane == k, idx, iout)
            work = jnp.where(work == cur, -jnp.inf, work)
        wout_ref[...] = wout / denom
        iout_ref[...] = iout


def kernel(x, weight):
    n_tokens = x.shape[0]
    grid = (n_tokens // _BLOCK, _KCH)
    wout, iout = pl.pallas_call(
        _gate_block,
        grid=grid,
        in_specs=[
            pl.BlockSpec((_BLOCK, _CDIM), lambda i, j: (i, j)),
            pl.BlockSpec((_E, _CDIM), lambda i, j: (0, j)),
        ],
        out_specs=[
            pl.BlockSpec((_BLOCK, _K), lambda i, j: (i, 0)),
            pl.BlockSpec((_BLOCK, _K), lambda i, j: (i, 0)),
        ],
        out_shape=[
            jax.ShapeDtypeStruct((n_tokens, _K), jnp.float32),
            jax.ShapeDtypeStruct((n_tokens, _K), jnp.int32),
        ],
        scratch_shapes=[pltpu.VMEM((_BLOCK, _E), jnp.float32)],
        compiler_params=pltpu.CompilerParams(
            dimension_semantics=("parallel", "arbitrary")),
    )(x, weight)
    return wout, iout


# speedup vs baseline: 1.4193x; 1.4193x over previous
import jax
import jax.numpy as jnp
from jax.experimental import pallas as pl
from jax.experimental.pallas import tpu as pltpu

_DIM = 4096
_E = 64
_K = 8
_BLOCK = 1024


def _gate_block(x_ref, w_ref, wout_ref, iout_ref):
    x = x_ref[...]
    w = w_ref[...]
    scores = jax.lax.dot_general(
        x, w, (((1,), (1,)), ((), ())),
        preferred_element_type=jnp.float32)
    m = jnp.max(scores, axis=1, keepdims=True)
    e = jnp.exp(scores - m)
    denom = jnp.sum(e, axis=1, keepdims=True)

    lane = jax.lax.broadcasted_iota(jnp.int32, e.shape, 1)
    bits = jax.lax.bitcast_convert_type(e, jnp.int32)
    key = jax.lax.bitcast_convert_type(
        (bits & ~(_E - 1)) | ((_E - 1) - lane), jnp.float32)

    out_lane = jax.lax.broadcasted_iota(jnp.int32, (e.shape[0], _K), 1)
    wout = jnp.zeros((e.shape[0], _K), jnp.float32)
    iout = jnp.zeros((e.shape[0], _K), jnp.int32)
    work = key
    for k in range(_K):
        cur = jnp.max(work, axis=1, keepdims=True)
        cur_bits = jax.lax.bitcast_convert_type(cur, jnp.int32)
        idx = (_E - 1) - (cur_bits & (_E - 1))
        val = jax.lax.bitcast_convert_type(cur_bits & ~(_E - 1), jnp.float32)
        wout = jnp.where(out_lane == k, val, wout)
        iout = jnp.where(out_lane == k, idx, iout)
        work = jnp.where(work == cur, -jnp.inf, work)
    wout_ref[...] = wout / denom
    iout_ref[...] = iout


def kernel(x, weight):
    n_tokens = x.shape[0]
    grid = (n_tokens // _BLOCK,)
    wout, iout = pl.pallas_call(
        _gate_block,
        grid=grid,
        in_specs=[
            pl.BlockSpec((_BLOCK, _DIM), lambda i: (i, 0)),
            pl.BlockSpec((_E, _DIM), lambda i: (0, 0)),
        ],
        out_specs=[
            pl.BlockSpec((_BLOCK, _K), lambda i: (i, 0)),
            pl.BlockSpec((_BLOCK, _K), lambda i: (i, 0)),
        ],
        out_shape=[
            jax.ShapeDtypeStruct((n_tokens, _K), jnp.float32),
            jax.ShapeDtypeStruct((n_tokens, _K), jnp.int32),
        ],
        compiler_params=pltpu.CompilerParams(
            dimension_semantics=("parallel",)),
    )(x, weight)
    return wout, iout
